# Initial kernel scaffold; baseline (speedup 1.0000x reference)
import jax
import jax.numpy as jnp
from jax.experimental import pallas as pl


def kernel(x, types, W, b):
    raise NotImplementedError("write your pallas kernel here")


# speedup vs baseline: 56.3783x; 56.3783x over previous
import functools

import jax
import jax.numpy as jnp
from jax import lax
from jax.experimental import pallas as pl
from jax.experimental.pallas import tpu as pltpu
from jax.experimental.pallas import tpu_sc as plsc

B = 2048
D_IN = 768
D_OUT = 768
G = 64
TM = 128
NT = B // TM
NS = NT + G - 1

SC_CORES = 2
SC_SUBCORES = 16
NW = SC_CORES * SC_SUBCORES
B_PER_W = B // NW


def _routing(types):
    perm = jnp.argsort(types, stable=True).astype(jnp.int32)
    inv_perm = jnp.zeros((B,), jnp.int32).at[perm].set(
        jnp.arange(B, dtype=jnp.int32))
    counts = jnp.bincount(types, length=G)
    ends = jnp.cumsum(counts)
    starts = ends - counts
    nonempty = counts > 0
    first_tile = starts // TM
    last_tile = (ends - 1) // TM
    tiles_g = jnp.where(nonempty, last_tile - first_tile + 1, 0)
    work_cum = jnp.cumsum(tiles_g)
    total = work_cum[G - 1]
    s_idx = jnp.arange(NS)
    gid = jnp.searchsorted(work_cum, s_idx, side="right")
    valid = s_idx < total
    gidc = jnp.minimum(gid, G - 1)
    wstart = work_cum[gidc] - tiles_g[gidc]
    tid = jnp.where(valid, first_tile[gidc] + (s_idx - wstart), NT - 1)
    row_lo = jnp.where(valid, starts[gidc], 0)
    row_hi = jnp.where(valid, ends[gidc], 0)
    gidc = jnp.where(valid, gidc, 0)
    return (perm, inv_perm, tid.astype(jnp.int32), gidc.astype(jnp.int32),
            row_lo.astype(jnp.int32), row_hi.astype(jnp.int32))


def _sc_gather(table, idx):
    n_rows, d = table.shape
    mesh = plsc.VectorSubcoreMesh(
        core_axis_name="c", subcore_axis_name="s",
        num_cores=SC_CORES, num_subcores=SC_SUBCORES)

    @functools.partial(
        pl.kernel,
        out_type=jax.ShapeDtypeStruct((B, d), table.dtype),
        mesh=mesh,
        scratch_types=[
            pltpu.VMEM((B_PER_W,), jnp.int32),
            pltpu.VMEM((B_PER_W, d), table.dtype),
            pltpu.SemaphoreType.DMA,
        ],
    )
    def k(table_hbm, idx_hbm, out_hbm, idx_v, rows_v, sem):
        wid = lax.axis_index("s") * SC_CORES + lax.axis_index("c")
        base = wid * B_PER_W
        pltpu.sync_copy(idx_hbm.at[pl.ds(base, B_PER_W)], idx_v)
        pltpu.async_copy(table_hbm.at[idx_v], rows_v, sem).wait()
        pltpu.sync_copy(rows_v, out_hbm.at[pl.ds(base, B_PER_W)])

    return k(table, idx)


def _grouped_body(tid_ref, gid_ref, lo_ref, hi_ref, x_ref, w_ref, b_ref,
                  o_ref):
    s = pl.program_id(0)
    t = tid_ref[s]
    rows = t * TM + lax.broadcasted_iota(jnp.int32, (TM, 1), 0)
    m = (rows >= lo_ref[s]) & (rows < hi_ref[s])
    xm = jnp.where(m, x_ref[...], 0.0)
    part = lax.dot_general(
        xm, w_ref[0], (((1,), (1,)), ((), ())),
        preferred_element_type=jnp.float32,
        precision=lax.Precision.HIGHEST)
    part = part + jnp.where(m, b_ref[0], 0.0)
    prev_t = tid_ref[jnp.maximum(s - 1, 0)]
    first = jnp.logical_or(s == 0, t != prev_t)

    @pl.when(first)
    def _():
        o_ref[...] = part

    @pl.when(jnp.logical_not(first))
    def _():
        o_ref[...] += part


def _grouped_matmul(xs, W, b, tid, gid, row_lo, row_hi):
    grid_spec = pltpu.PrefetchScalarGridSpec(
        num_scalar_prefetch=4,
        grid=(NS,),
        in_specs=[
            pl.BlockSpec((TM, D_IN), lambda s, t, g, lo, hi: (t[s], 0)),
            pl.BlockSpec((1, D_OUT, D_IN), lambda s, t, g, lo, hi: (g[s], 0, 0)),
            pl.BlockSpec((1, 1, D_OUT), lambda s, t, g, lo, hi: (g[s], 0, 0)),
        ],
        out_specs=pl.BlockSpec((TM, D_OUT), lambda s, t, g, lo, hi: (t[s], 0)),
    )
    return pl.pallas_call(
        _grouped_body,
        grid_spec=grid_spec,
        out_shape=jax.ShapeDtypeStruct((B, D_OUT), jnp.float32),
    )(tid, gid, row_lo, row_hi, xs, W, b.reshape(G, 1, D_OUT))


def kernel(x, types, W, b):
    perm, inv_perm, tid, gid, row_lo, row_hi = _routing(types)
    xs = _sc_gather(x, perm)
    ys = _grouped_matmul(xs, W, b, tid, gid, row_lo, row_hi)
    return _sc_gather(ys, inv_perm)
